# baseline (speedup 1.0000x reference)
import jax
import jax.numpy as jnp
from jax import lax
from jax.experimental import pallas as pl
from jax.experimental.pallas import tpu as pltpu
from jax.experimental.pallas import tpu_sc as plsc

B = 16384
D = 64
NC = 2
NS = 16
NW = NC * NS
BPW = B // NW
NCHUNK = 4
CHUNK = BPW // NCHUNK
NG = BPW // 16


def _rsqrt(x):
    i = plsc.bitcast(x, jnp.int32)
    i = jnp.int32(0x5F3759DF) - (i >> 1)
    y = plsc.bitcast(i, jnp.float32)
    for _ in range(3):
        y = y * (1.5 - 0.5 * x * y * y)
    return y


def _body(uidx_h, midx_h, gidx_h, yidx_h,
          uemb_h, memb_h, gemb_h, yemb_h, ubias_h, mbias_h,
          out_h,
          uidx_v, midx_v, gidx_v, yidx_v,
          urows_v, mrows_v, gtbl_v, ytbl_v,
          ub_v, mb_v, out_v, sem):
    wid = lax.axis_index("s") * NC + lax.axis_index("c")

    pltpu.sync_copy(uidx_h.at[wid], uidx_v)
    pltpu.sync_copy(midx_h.at[wid], midx_v)
    pltpu.sync_copy(gidx_h.at[wid], gidx_v)
    pltpu.sync_copy(yidx_h.at[wid], yidx_v)

    descs = [
        pltpu.async_copy(gemb_h, gtbl_v, sem),
        pltpu.async_copy(yemb_h, ytbl_v, sem),
    ]
    for j in range(NCHUNK):
        dst = pl.ds(j * CHUNK, CHUNK)
        descs.append(pltpu.async_copy(uemb_h.at[uidx_v.at[j]],
                                      urows_v.at[dst], sem))
        descs.append(pltpu.async_copy(memb_h.at[midx_v.at[j]],
                                      mrows_v.at[dst], sem))
        descs.append(pltpu.async_copy(ubias_h.at[uidx_v.at[j]],
                                      ub_v.at[dst], sem))
        descs.append(pltpu.async_copy(mbias_h.at[midx_v.at[j]],
                                      mb_v.at[dst], sem))
    for dsc in descs:
        dsc.wait()

    def group(g, _):
        rows = g * 16 + lax.iota(jnp.int32, 16)
        giv = plsc.load_gather(gidx_v, [rows])
        yiv = plsc.load_gather(yidx_v, [rows])

        def col(c, carry):
            s_um, s_uu, s_mm = carry
            colv = jnp.broadcast_to(c, (16,))
            u = plsc.load_gather(urows_v, [rows, colv])
            mv = plsc.load_gather(mrows_v, [rows, colv])
            gv = plsc.load_gather(gtbl_v, [giv, colv])
            yv = plsc.load_gather(ytbl_v, [yiv, colv])
            m = mv + gv + yv
            return (s_um + u * m, s_uu + u * u, s_mm + m * m)

        zeros = jnp.zeros((16,), jnp.float32)
        s_um, s_uu, s_mm = lax.fori_loop(
            0, D, col, (zeros, zeros, zeros), unroll=8)

        inv = _rsqrt(jnp.maximum(s_uu, 1e-16)) * _rsqrt(jnp.maximum(s_mm, 1e-16))
        ub = plsc.load_gather(ub_v, [rows])
        mb = plsc.load_gather(mb_v, [rows])
        pred = s_um * inv * 2.5 + 2.75 + ub + mb
        plsc.store_scatter(out_v, [rows], pred)
        return 0

    lax.fori_loop(0, NG, group, 0)

    base = pl.multiple_of(wid * BPW, BPW)
    pltpu.sync_copy(out_v, out_h.at[pl.ds(base, BPW)])


def kernel(user_idx, movie_idx, genre_idx, year_idx,
           user_embeds, movie_embeds, genre_embeds, year_embeds,
           user_biases, movie_biases):
    mesh = plsc.VectorSubcoreMesh(core_axis_name="c", subcore_axis_name="s",
                                  num_cores=NC, num_subcores=NS)
    f32 = jnp.float32
    i32 = jnp.int32
    k = pl.kernel(
        _body,
        out_type=jax.ShapeDtypeStruct((B,), f32),
        mesh=mesh,
        compiler_params=pltpu.CompilerParams(needs_layout_passes=False,
                                             use_tc_tiling_on_sc=False),
        scratch_types=[
            pltpu.VMEM((NCHUNK, CHUNK), i32),
            pltpu.VMEM((NCHUNK, CHUNK), i32),
            pltpu.VMEM((BPW,), i32),
            pltpu.VMEM((BPW,), i32),
            pltpu.VMEM((BPW, D), f32),
            pltpu.VMEM((BPW, D), f32),
            pltpu.VMEM((20, D), f32),
            pltpu.VMEM((100, D), f32),
            pltpu.VMEM((BPW,), f32),
            pltpu.VMEM((BPW,), f32),
            pltpu.VMEM((BPW,), f32),
            pltpu.SemaphoreType.DMA,
        ],
    )
    return k(user_idx.astype(i32).reshape(NW, NCHUNK, CHUNK),
             movie_idx.astype(i32).reshape(NW, NCHUNK, CHUNK),
             genre_idx.astype(i32).reshape(NW, BPW),
             year_idx.astype(i32).reshape(NW, BPW),
             user_embeds, movie_embeds, genre_embeds, year_embeds,
             user_biases.reshape(-1), movie_biases.reshape(-1))
